# baseline (speedup 1.0000x reference)
import jax
import jax.numpy as jnp
from jax import lax
from jax.experimental import pallas as pl
from jax.experimental.pallas import tpu as pltpu
from jax.experimental.pallas import tpu_sc as plsc

N_NODES = 100000
N_EDGES = 6400000
NC = 2
NS = 16
NW = NC * NS
EPT = N_EDGES // NW
BLK = 10000
NB = EPT // BLK
NPAD = 102400
ZB = NPAD // NS
W = 16384


def _sc_body(x_hbm, ptrs_hbm, csr_hbm, out_hbm,
             xs, acc, acc_loc, fidx,
             ptr0, ptr1, csr0, csr1, val0, val1, zb, cfirst, clast,
             lds0, lds1, gsm0, gsm1, pre):
    c = lax.axis_index("c")
    s = lax.axis_index("s")
    ptr_v = (ptr0, ptr1)
    csr_v = (csr0, csr1)
    val_v = (val0, val1)
    lds = (lds0, lds1)
    gsm = (gsm0, gsm1)

    base = (c * NS + s) * EPT

    pltpu.async_copy(csr_hbm.at[pl.ds(base, 16)], cfirst, pre)
    pltpu.async_copy(csr_hbm.at[pl.ds(base + EPT - 16, 16)], clast, pre)

    def zfill(j, carry):
        zb[pl.ds(j * 16, 16)] = jnp.zeros((16,), jnp.float32)
        return carry
    lax.fori_loop(0, ZB // 16, zfill, 0)
    pltpu.sync_copy(zb, acc.at[pl.ds(s * ZB, ZB)])

    @pl.when(s == 0)
    def _():
        pltpu.sync_copy(x_hbm, xs)

    pltpu.make_async_copy(csr_hbm.at[pl.ds(base, 16)], cfirst, pre).wait()
    pltpu.make_async_copy(
        csr_hbm.at[pl.ds(base + EPT - 16, 16)], clast, pre).wait()
    lo = jnp.bitwise_and(jnp.min(cfirst[pl.ds(0, 16)]), jnp.int32(-8))
    lo = jnp.minimum(lo, jnp.int32(N_NODES - W))
    hi = jnp.max(clast[pl.ds(0, 16)])
    fast = (hi - lo) < W

    @pl.when(fast)
    def _():
        def wfill(j, carry):
            acc_loc[pl.ds(j * 16, 16)] = jnp.zeros((16,), jnp.float32)
            return carry
        lax.fori_loop(0, W // 16, wfill, 0)

    plsc.subcore_barrier()

    def issue_loads(b, par):
        off = base + b * BLK
        pltpu.async_copy(ptrs_hbm.at[pl.ds(off, BLK)], ptr_v[par], lds[par])
        pltpu.async_copy(csr_hbm.at[pl.ds(off, BLK)], csr_v[par], lds[par])

    def wait_loads(b, par):
        off = base + b * BLK
        pltpu.make_async_copy(
            ptrs_hbm.at[pl.ds(off, BLK)], ptr_v[par], lds[par]).wait()
        pltpu.make_async_copy(
            csr_hbm.at[pl.ds(off, BLK)], csr_v[par], lds[par]).wait()

    issue_loads(0, 0)
    issue_loads(1, 1)
    wait_loads(0, 0)
    pltpu.async_copy(xs.at[ptr_v[0]], val_v[0], gsm[0])

    def step(b, par):
        nxt = 1 - par
        pltpu.make_async_copy(xs.at[ptr_v[par]], val_v[par], gsm[par]).wait()
        @pl.when(b + 1 < NB)
        def _():
            wait_loads(b + 1, nxt)
            pltpu.async_copy(xs.at[ptr_v[nxt]], val_v[nxt], gsm[nxt])

        @pl.when(fast)
        def _():
            def chunk(j, carry):
                cvec = csr_v[par][pl.ds(j * 16, 16)]
                vvec = val_v[par][pl.ds(j * 16, 16)]
                plsc.addupdate_scatter(acc_loc, [cvec - lo], vvec)
                return carry
            lax.fori_loop(0, BLK // 16, chunk, 0)

        @pl.when(jnp.logical_not(fast))
        def _():
            pltpu.sync_copy(val_v[par], acc.at[csr_v[par]], add=True)

        @pl.when(b + 2 < NB)
        def _():
            issue_loads(b + 2, par)

    def pair(g, carry):
        step(2 * g, 0)
        step(2 * g + 1, 1)
        return carry
    lax.fori_loop(0, NB // 2, pair, 0)

    @pl.when(fast)
    def _():
        iota16 = lax.iota(jnp.int32, 16)

        def ffill(j, carry):
            fidx[pl.ds(j * 16, 16)] = iota16 + (lo + j * 16)
            return carry
        lax.fori_loop(0, W // 16, ffill, 0)
        pltpu.sync_copy(acc_loc, acc.at[fidx], add=True)

    plsc.subcore_barrier()

    pltpu.sync_copy(acc.at[pl.ds(s * ZB, ZB)],
                    out_hbm.at[c, pl.ds(s * ZB, ZB)])


@jax.jit
def _sc_scatter(x, ptrs, csr):
    mesh = plsc.VectorSubcoreMesh(core_axis_name="c", subcore_axis_name="s")
    f = pl.kernel(
        _sc_body, mesh=mesh,
        compiler_params=pltpu.CompilerParams(needs_layout_passes=False),
        out_type=jax.ShapeDtypeStruct((NC, NPAD), jnp.float32),
        scratch_types=[
            pltpu.MemorySpace.VMEM_SHARED((N_NODES,), jnp.float32),
            pltpu.MemorySpace.VMEM_SHARED((NPAD,), jnp.float32),
            pltpu.VMEM((W,), jnp.float32),
            pltpu.VMEM((W,), jnp.int32),
            pltpu.VMEM((BLK,), jnp.int32),
            pltpu.VMEM((BLK,), jnp.int32),
            pltpu.VMEM((BLK,), jnp.int32),
            pltpu.VMEM((BLK,), jnp.int32),
            pltpu.VMEM((BLK,), jnp.float32),
            pltpu.VMEM((BLK,), jnp.float32),
            pltpu.VMEM((ZB,), jnp.float32),
            pltpu.VMEM((16,), jnp.int32),
            pltpu.VMEM((16,), jnp.int32),
            pltpu.SemaphoreType.DMA,
            pltpu.SemaphoreType.DMA,
            pltpu.SemaphoreType.DMA,
            pltpu.SemaphoreType.DMA,
            pltpu.SemaphoreType.DMA,
        ],
    )
    return f(x, ptrs, csr)


def _combine_body(p_ref, o_ref):
    o_ref[...] = p_ref[0] + p_ref[1]


@jax.jit
def _combine(partials):
    p = partials.reshape(NC, NPAD // 128, 128)
    out = pl.pallas_call(
        _combine_body,
        out_shape=jax.ShapeDtypeStruct((NPAD // 128, 128), jnp.float32),
    )(p)
    return out.reshape(-1)[:N_NODES]


def kernel(x, ptrs, csr):
    partials = _sc_scatter(x, ptrs, csr)
    return _combine(partials)


# speedup vs baseline: 1.1894x; 1.1894x over previous
import jax
import jax.numpy as jnp
from jax import lax
from jax.experimental import pallas as pl
from jax.experimental.pallas import tpu as pltpu
from jax.experimental.pallas import tpu_sc as plsc

N_NODES = 100000
N_EDGES = 6400000
NC = 2
NS = 16
NW = NC * NS
EPT = N_EDGES // NW
BLK = 10000
NB = EPT // BLK
NPAD = 102400
ZB = NPAD // NS
W = 16384

_GDN = lax.GatherDimensionNumbers(
    offset_dims=(), collapsed_slice_dims=(0,), start_index_map=(0,))


def _vgather(vec, idx):
    return lax.gather(vec, idx.reshape(16, 1), _GDN, (1,),
                      mode=lax.GatherScatterMode.PROMISE_IN_BOUNDS)


def _sc_body(x_hbm, ptrs_hbm, csr_hbm, out_hbm,
             xs, acc, acc_loc, fidx,
             ptr0, ptr1, csr0, csr1, val0, val1, zb, cfirst, clast,
             lds0, lds1, gsm0, gsm1, pre):
    c = lax.axis_index("c")
    s = lax.axis_index("s")
    ptr_v = (ptr0, ptr1)
    csr_v = (csr0, csr1)
    val_v = (val0, val1)
    lds = (lds0, lds1)
    gsm = (gsm0, gsm1)

    base = (c * NS + s) * EPT

    pltpu.async_copy(csr_hbm.at[pl.ds(base, 16)], cfirst, pre)
    pltpu.async_copy(csr_hbm.at[pl.ds(base + EPT - 16, 16)], clast, pre)

    def zfill(j, carry):
        zb[pl.ds(j * 16, 16)] = jnp.zeros((16,), jnp.float32)
        return carry
    lax.fori_loop(0, ZB // 16, zfill, 0)
    pltpu.sync_copy(zb, acc.at[pl.ds(s * ZB, ZB)])

    @pl.when(s == 0)
    def _():
        pltpu.sync_copy(x_hbm, xs)

    pltpu.make_async_copy(csr_hbm.at[pl.ds(base, 16)], cfirst, pre).wait()
    pltpu.make_async_copy(
        csr_hbm.at[pl.ds(base + EPT - 16, 16)], clast, pre).wait()
    lo = jnp.bitwise_and(jnp.min(cfirst[pl.ds(0, 16)]), jnp.int32(-8))
    lo = jnp.minimum(lo, jnp.int32(N_NODES - W))
    hi = jnp.max(clast[pl.ds(0, 16)])
    fast = (hi - lo) < W

    @pl.when(fast)
    def _():
        def wfill(j, carry):
            acc_loc[pl.ds(j * 16, 16)] = jnp.zeros((16,), jnp.float32)
            return carry
        lax.fori_loop(0, W // 16, wfill, 0)

    plsc.subcore_barrier()

    def issue_loads(b, par):
        off = base + b * BLK
        pltpu.async_copy(ptrs_hbm.at[pl.ds(off, BLK)], ptr_v[par], lds[par])
        pltpu.async_copy(csr_hbm.at[pl.ds(off, BLK)], csr_v[par], lds[par])

    def wait_loads(b, par):
        off = base + b * BLK
        pltpu.make_async_copy(
            ptrs_hbm.at[pl.ds(off, BLK)], ptr_v[par], lds[par]).wait()
        pltpu.make_async_copy(
            csr_hbm.at[pl.ds(off, BLK)], csr_v[par], lds[par]).wait()

    issue_loads(0, 0)
    issue_loads(1, 1)
    wait_loads(0, 0)
    pltpu.async_copy(xs.at[ptr_v[0]], val_v[0], gsm[0])

    iota16 = lax.iota(jnp.int32, 16)
    shl_idx = jnp.minimum(iota16 + 1, 15)
    sh_idx = tuple(jnp.maximum(iota16 - d, 0) for d in (1, 2, 4, 8))

    def step(b, par):
        nxt = 1 - par
        pltpu.make_async_copy(xs.at[ptr_v[par]], val_v[par], gsm[par]).wait()
        @pl.when(b + 1 < NB)
        def _():
            wait_loads(b + 1, nxt)
            pltpu.async_copy(xs.at[ptr_v[nxt]], val_v[nxt], gsm[nxt])

        @pl.when(fast)
        def _():
            def chunk(j, carry):
                cvec = csr_v[par][pl.ds(j * 16, 16)]
                vvec = val_v[par][pl.ds(j * 16, 16)]
                sacc = vvec
                for d, sidx in zip((1, 2, 4, 8), sh_idx):
                    same = jnp.logical_and(iota16 >= d,
                                           cvec == _vgather(cvec, sidx))
                    sacc = sacc + jnp.where(same, _vgather(sacc, sidx), 0.0)
                ends = jnp.logical_or(cvec != _vgather(cvec, shl_idx),
                                      iota16 == 15)
                plsc.addupdate_scatter(acc_loc, [cvec - lo], sacc, mask=ends)
                return carry
            lax.fori_loop(0, BLK // 16, chunk, 0)

        @pl.when(jnp.logical_not(fast))
        def _():
            pltpu.sync_copy(val_v[par], acc.at[csr_v[par]], add=True)

        @pl.when(b + 2 < NB)
        def _():
            issue_loads(b + 2, par)

    def pair(g, carry):
        step(2 * g, 0)
        step(2 * g + 1, 1)
        return carry
    lax.fori_loop(0, NB // 2, pair, 0)

    @pl.when(fast)
    def _():
        def ffill(j, carry):
            fidx[pl.ds(j * 16, 16)] = iota16 + (lo + j * 16)
            return carry
        lax.fori_loop(0, W // 16, ffill, 0)
        pltpu.sync_copy(acc_loc, acc.at[fidx], add=True)

    plsc.subcore_barrier()

    pltpu.sync_copy(acc.at[pl.ds(s * ZB, ZB)],
                    out_hbm.at[c, pl.ds(s * ZB, ZB)])


@jax.jit
def _sc_scatter(x, ptrs, csr):
    mesh = plsc.VectorSubcoreMesh(core_axis_name="c", subcore_axis_name="s")
    f = pl.kernel(
        _sc_body, mesh=mesh,
        compiler_params=pltpu.CompilerParams(needs_layout_passes=False),
        out_type=jax.ShapeDtypeStruct((NC, NPAD), jnp.float32),
        scratch_types=[
            pltpu.MemorySpace.VMEM_SHARED((N_NODES,), jnp.float32),
            pltpu.MemorySpace.VMEM_SHARED((NPAD,), jnp.float32),
            pltpu.VMEM((W,), jnp.float32),
            pltpu.VMEM((W,), jnp.int32),
            pltpu.VMEM((BLK,), jnp.int32),
            pltpu.VMEM((BLK,), jnp.int32),
            pltpu.VMEM((BLK,), jnp.int32),
            pltpu.VMEM((BLK,), jnp.int32),
            pltpu.VMEM((BLK,), jnp.float32),
            pltpu.VMEM((BLK,), jnp.float32),
            pltpu.VMEM((ZB,), jnp.float32),
            pltpu.VMEM((16,), jnp.int32),
            pltpu.VMEM((16,), jnp.int32),
            pltpu.SemaphoreType.DMA,
            pltpu.SemaphoreType.DMA,
            pltpu.SemaphoreType.DMA,
            pltpu.SemaphoreType.DMA,
            pltpu.SemaphoreType.DMA,
        ],
    )
    return f(x, ptrs, csr)


def _combine_body(p_ref, o_ref):
    o_ref[...] = p_ref[0] + p_ref[1]


@jax.jit
def _combine(partials):
    p = partials.reshape(NC, NPAD // 128, 128)
    out = pl.pallas_call(
        _combine_body,
        out_shape=jax.ShapeDtypeStruct((NPAD // 128, 128), jnp.float32),
    )(p)
    return out.reshape(-1)[:N_NODES]


def kernel(x, ptrs, csr):
    partials = _sc_scatter(x, ptrs, csr)
    return _combine(partials)


# speedup vs baseline: 1.2129x; 1.0198x over previous
import jax
import jax.numpy as jnp
from jax import lax
from jax.experimental import pallas as pl
from jax.experimental.pallas import tpu as pltpu
from jax.experimental.pallas import tpu_sc as plsc

N_NODES = 100000
N_EDGES = 6400000
NC = 2
NS = 16
NW = NC * NS
EPT = N_EDGES // NW
BLK = 10000
NB = EPT // BLK
NPAD = 102400
ZB = NPAD // NS
W = 16384

_GDN = lax.GatherDimensionNumbers(
    offset_dims=(), collapsed_slice_dims=(0,), start_index_map=(0,))


def _vgather(vec, idx):
    return lax.gather(vec, idx.reshape(16, 1), _GDN, (1,),
                      mode=lax.GatherScatterMode.PROMISE_IN_BOUNDS)


def _sc_body(x_hbm, ptrs_hbm, csr_hbm, out_hbm,
             xs, acc, acc_loc, fidx,
             ptr0, ptr1, csr0, csr1, val0, val1, zb, cfirst, clast,
             lds0, lds1, gsm0, gsm1, pre):
    c = lax.axis_index("c")
    s = lax.axis_index("s")
    ptr_v = (ptr0, ptr1)
    csr_v = (csr0, csr1)
    val_v = (val0, val1)
    lds = (lds0, lds1)
    gsm = (gsm0, gsm1)

    base = (c * NS + s) * EPT

    pltpu.async_copy(csr_hbm.at[pl.ds(base, 16)], cfirst, pre)
    pltpu.async_copy(csr_hbm.at[pl.ds(base + EPT - 16, 16)], clast, pre)

    def zfill(j, carry):
        zb[pl.ds(j * 16, 16)] = jnp.zeros((16,), jnp.float32)
        return carry
    lax.fori_loop(0, ZB // 16, zfill, 0)
    pltpu.sync_copy(zb, acc.at[pl.ds(s * ZB, ZB)])

    @pl.when(s == 0)
    def _():
        pltpu.sync_copy(x_hbm, xs)

    pltpu.make_async_copy(csr_hbm.at[pl.ds(base, 16)], cfirst, pre).wait()
    pltpu.make_async_copy(
        csr_hbm.at[pl.ds(base + EPT - 16, 16)], clast, pre).wait()
    lo = jnp.bitwise_and(jnp.min(cfirst[pl.ds(0, 16)]), jnp.int32(-8))
    lo = jnp.minimum(lo, jnp.int32(N_NODES - W))
    hi = jnp.max(clast[pl.ds(0, 16)])
    fast = (hi - lo) < W

    @pl.when(fast)
    def _():
        def wfill(j, carry):
            acc_loc[pl.ds(j * 16, 16)] = jnp.zeros((16,), jnp.float32)
            return carry
        lax.fori_loop(0, W // 16, wfill, 0)

    plsc.subcore_barrier()

    def issue_loads(b, par):
        off = base + b * BLK
        pltpu.async_copy(ptrs_hbm.at[pl.ds(off, BLK)], ptr_v[par], lds[par])
        pltpu.async_copy(csr_hbm.at[pl.ds(off, BLK)], csr_v[par], lds[par])

    def wait_loads(b, par):
        off = base + b * BLK
        pltpu.make_async_copy(
            ptrs_hbm.at[pl.ds(off, BLK)], ptr_v[par], lds[par]).wait()
        pltpu.make_async_copy(
            csr_hbm.at[pl.ds(off, BLK)], csr_v[par], lds[par]).wait()

    issue_loads(0, 0)
    issue_loads(1, 1)
    wait_loads(0, 0)
    pltpu.async_copy(xs.at[ptr_v[0]], val_v[0], gsm[0])

    iota16 = lax.iota(jnp.int32, 16)
    shl_idx = jnp.minimum(iota16 + 1, 15)
    sh_idx = tuple(jnp.maximum(iota16 - d, 0) for d in (1, 2, 4, 8))

    def step(b, par):
        nxt = 1 - par
        pltpu.make_async_copy(xs.at[ptr_v[par]], val_v[par], gsm[par]).wait()
        @pl.when(b + 1 < NB)
        def _():
            wait_loads(b + 1, nxt)
            pltpu.async_copy(xs.at[ptr_v[nxt]], val_v[nxt], gsm[nxt])

        @pl.when(fast)
        def _():
            def chunk(jj, carry):
                for k in range(5):
                    j = jj * 5 + k
                    cvec = csr_v[par][pl.ds(j * 16, 16)]
                    vvec = val_v[par][pl.ds(j * 16, 16)]
                    sacc = vvec
                    for d, sidx in zip((1, 2, 4, 8), sh_idx):
                        same = jnp.logical_and(iota16 >= d,
                                               cvec == _vgather(cvec, sidx))
                        sacc = sacc + jnp.where(same,
                                                _vgather(sacc, sidx), 0.0)
                    ends = jnp.logical_or(cvec != _vgather(cvec, shl_idx),
                                          iota16 == 15)
                    plsc.addupdate_scatter(acc_loc, [cvec - lo], sacc,
                                           mask=ends)
                return carry
            lax.fori_loop(0, BLK // 80, chunk, 0)

        @pl.when(jnp.logical_not(fast))
        def _():
            pltpu.sync_copy(val_v[par], acc.at[csr_v[par]], add=True)

        @pl.when(b + 2 < NB)
        def _():
            issue_loads(b + 2, par)

    def pair(g, carry):
        step(2 * g, 0)
        step(2 * g + 1, 1)
        return carry
    lax.fori_loop(0, NB // 2, pair, 0)

    @pl.when(fast)
    def _():
        def ffill(j, carry):
            fidx[pl.ds(j * 16, 16)] = iota16 + (lo + j * 16)
            return carry
        lax.fori_loop(0, W // 16, ffill, 0)
        pltpu.sync_copy(acc_loc, acc.at[fidx], add=True)

    plsc.subcore_barrier()

    pltpu.sync_copy(acc.at[pl.ds(s * ZB, ZB)],
                    out_hbm.at[c, pl.ds(s * ZB, ZB)])


@jax.jit
def _sc_scatter(x, ptrs, csr):
    mesh = plsc.VectorSubcoreMesh(core_axis_name="c", subcore_axis_name="s")
    f = pl.kernel(
        _sc_body, mesh=mesh,
        compiler_params=pltpu.CompilerParams(needs_layout_passes=False),
        out_type=jax.ShapeDtypeStruct((NC, NPAD), jnp.float32),
        scratch_types=[
            pltpu.MemorySpace.VMEM_SHARED((N_NODES,), jnp.float32),
            pltpu.MemorySpace.VMEM_SHARED((NPAD,), jnp.float32),
            pltpu.VMEM((W,), jnp.float32),
            pltpu.VMEM((W,), jnp.int32),
            pltpu.VMEM((BLK,), jnp.int32),
            pltpu.VMEM((BLK,), jnp.int32),
            pltpu.VMEM((BLK,), jnp.int32),
            pltpu.VMEM((BLK,), jnp.int32),
            pltpu.VMEM((BLK,), jnp.float32),
            pltpu.VMEM((BLK,), jnp.float32),
            pltpu.VMEM((ZB,), jnp.float32),
            pltpu.VMEM((16,), jnp.int32),
            pltpu.VMEM((16,), jnp.int32),
            pltpu.SemaphoreType.DMA,
            pltpu.SemaphoreType.DMA,
            pltpu.SemaphoreType.DMA,
            pltpu.SemaphoreType.DMA,
            pltpu.SemaphoreType.DMA,
        ],
    )
    return f(x, ptrs, csr)


def _combine_body(p_ref, o_ref):
    o_ref[...] = p_ref[0] + p_ref[1]


@jax.jit
def _combine(partials):
    p = partials.reshape(NC, NPAD // 128, 128)
    out = pl.pallas_call(
        _combine_body,
        out_shape=jax.ShapeDtypeStruct((NPAD // 128, 128), jnp.float32),
    )(p)
    return out.reshape(-1)[:N_NODES]


def kernel(x, ptrs, csr):
    partials = _sc_scatter(x, ptrs, csr)
    return _combine(partials)


# speedup vs baseline: 1.2923x; 1.0654x over previous
import jax
import jax.numpy as jnp
from jax import lax
from jax.experimental import pallas as pl
from jax.experimental.pallas import tpu as pltpu
from jax.experimental.pallas import tpu_sc as plsc

N_NODES = 100000
N_EDGES = 6400000
NC = 2
NS = 16
NW = NC * NS
EPT = N_EDGES // NW
BLK = 10000
NB = EPT // BLK
NPAD = 102400
ZB = NPAD // NS


def _sc_body(x_hbm, ptrs_hbm, csr_hbm, out_hbm,
             xs, acc,
             ptr0, ptr1, csr0, csr1, csr2, val0, val1, val2, zb,
             lds0, lds1, lds2, gsm0, gsm1, gsm2, ssm0, ssm1, ssm2):
    c = lax.axis_index("c")
    s = lax.axis_index("s")
    ptr_v = (ptr0, ptr1)
    csr_v = (csr0, csr1, csr2)
    val_v = (val0, val1, val2)
    lds = (lds0, lds1, lds2)
    gsm = (gsm0, gsm1, gsm2)
    ssm = (ssm0, ssm1, ssm2)

    def zfill(j, carry):
        zb[pl.ds(j * 16, 16)] = jnp.zeros((16,), jnp.float32)
        return carry
    lax.fori_loop(0, ZB // 16, zfill, 0)
    pltpu.sync_copy(zb, acc.at[pl.ds(s * ZB, ZB)])

    @pl.when(s == 0)
    def _():
        pltpu.sync_copy(x_hbm, xs)

    plsc.subcore_barrier()

    base = (c * NS + s) * EPT

    def issue_loads(b, p2, p3):
        off = base + b * BLK
        pltpu.async_copy(ptrs_hbm.at[pl.ds(off, BLK)], ptr_v[p2], lds[p3])
        pltpu.async_copy(csr_hbm.at[pl.ds(off, BLK)], csr_v[p3], lds[p3])

    def wait_loads(b, p2, p3):
        off = base + b * BLK
        pltpu.make_async_copy(
            ptrs_hbm.at[pl.ds(off, BLK)], ptr_v[p2], lds[p3]).wait()
        pltpu.make_async_copy(
            csr_hbm.at[pl.ds(off, BLK)], csr_v[p3], lds[p3]).wait()

    issue_loads(0, 0, 0)
    issue_loads(1, 1, 1)
    wait_loads(0, 0, 0)
    pltpu.async_copy(xs.at[ptr_v[0]], val_v[0], gsm[0])

    def step(b, p2, p3):
        n2 = 1 - p2
        n3 = (p3 + 1) % 3
        w3 = (p3 + 2) % 3
        pltpu.make_async_copy(xs.at[ptr_v[p2]], val_v[p3], gsm[p3]).wait()
        @pl.when(b + 1 < NB)
        def _():
            wait_loads(b + 1, n2, n3)
            pltpu.async_copy(xs.at[ptr_v[n2]], val_v[n3], gsm[n3])
        @pl.when(b >= 1)
        def _():
            pltpu.make_async_copy(val_v[w3], acc.at[csr_v[w3]],
                                  ssm[w3]).wait()
        pltpu.async_copy(val_v[p3], acc.at[csr_v[p3]], ssm[p3], add=True)
        @pl.when(b + 2 < NB)
        def _():
            issue_loads(b + 2, p2, w3)

    def group(g, carry):
        b0 = 6 * g
        for k, (p2, p3) in enumerate(
                [(0, 0), (1, 1), (0, 2), (1, 0), (0, 1), (1, 2)]):
            step(b0 + k, p2, p3)
        return carry
    lax.fori_loop(0, NB // 6, group, 0)
    step(18, 0, 0)
    step(19, 1, 1)
    pltpu.make_async_copy(val_v[1], acc.at[csr_v[1]], ssm[1]).wait()

    plsc.subcore_barrier()

    pltpu.sync_copy(acc.at[pl.ds(s * ZB, ZB)],
                    out_hbm.at[c, pl.ds(s * ZB, ZB)])


@jax.jit
def _sc_scatter(x, ptrs, csr):
    mesh = plsc.VectorSubcoreMesh(core_axis_name="c", subcore_axis_name="s")
    f = pl.kernel(
        _sc_body, mesh=mesh,
        out_type=jax.ShapeDtypeStruct((NC, NPAD), jnp.float32),
        scratch_types=[
            pltpu.MemorySpace.VMEM_SHARED((N_NODES,), jnp.float32),
            pltpu.MemorySpace.VMEM_SHARED((NPAD,), jnp.float32),
            pltpu.VMEM((BLK,), jnp.int32),
            pltpu.VMEM((BLK,), jnp.int32),
            pltpu.VMEM((BLK,), jnp.int32),
            pltpu.VMEM((BLK,), jnp.int32),
            pltpu.VMEM((BLK,), jnp.int32),
            pltpu.VMEM((BLK,), jnp.float32),
            pltpu.VMEM((BLK,), jnp.float32),
            pltpu.VMEM((BLK,), jnp.float32),
            pltpu.VMEM((ZB,), jnp.float32),
            pltpu.SemaphoreType.DMA,
            pltpu.SemaphoreType.DMA,
            pltpu.SemaphoreType.DMA,
            pltpu.SemaphoreType.DMA,
            pltpu.SemaphoreType.DMA,
            pltpu.SemaphoreType.DMA,
            pltpu.SemaphoreType.DMA,
            pltpu.SemaphoreType.DMA,
            pltpu.SemaphoreType.DMA,
        ],
    )
    return f(x, ptrs, csr)


def _combine_body(p_ref, o_ref):
    o_ref[...] = p_ref[0] + p_ref[1]


@jax.jit
def _combine(partials):
    p = partials.reshape(NC, NPAD // 128, 128)
    out = pl.pallas_call(
        _combine_body,
        out_shape=jax.ShapeDtypeStruct((NPAD // 128, 128), jnp.float32),
    )(p)
    return out.reshape(-1)[:N_NODES]


def kernel(x, ptrs, csr):
    partials = _sc_scatter(x, ptrs, csr)
    return _combine(partials)
